# baseline (speedup 1.0000x reference)
import functools

import jax
import jax.numpy as jnp
from jax import lax
from jax.experimental import pallas as pl
from jax.experimental.pallas import tpu as pltpu
from jax.experimental.pallas import tpu_sc as plsc

_K = 16
_C = 512
_B = 8192
_NC = 2
_NS = 16
_NW = _NC * _NS
_BSC = 2048
_BTC = _B - _BSC
_RPW = _BSC // _NW
_CHUNK = 32
_NCHUNK = _RPW // _CHUNK
_EXT = _C + 16
_L = 16


def _sc_body(x_hbm, s_hbm, out_hbm, xbuf, sbuf, acc, sem):
    cid = lax.axis_index("c")
    sid = lax.axis_index("s")
    wid = sid * _NC + cid

    pltpu.sync_copy(s_hbm.at[pl.ds(wid * _RPW, _RPW)], sbuf)

    zeros = jnp.zeros((_L,), jnp.float32)

    def _zero(i, carry):
        acc[lax.div(i, _EXT // _L), pl.ds(lax.rem(i, _EXT // _L) * _L, _L)] = zeros
        return carry

    lax.fori_loop(0, _K * (_EXT // _L), _zero, 0)

    iota = lax.iota(jnp.int32, _L)

    def _issue(t):
        r0 = wid * _RPW + t * _CHUNK
        pltpu.async_copy(x_hbm.at[pl.ds(r0, _CHUNK), :],
                         xbuf.at[lax.rem(t, 2)], sem)

    def _drain(t):
        r0 = wid * _RPW + t * _CHUNK
        pltpu.make_async_copy(x_hbm.at[pl.ds(r0, _CHUNK), :],
                              xbuf.at[lax.rem(t, 2)], sem).wait()

    _issue(jnp.int32(0))

    def _chunk(t, carry):
        @pl.when(t + 1 < _NCHUNK)
        def _():
            _issue(t + 1)

        _drain(t)
        par = lax.rem(t, 2)

        @plsc.parallel_loop(0, _CHUNK, unroll=4)
        def _row(r):
            a0 = jnp.zeros((_L,), jnp.float32)
            a1 = jnp.zeros((_L,), jnp.float32)
            a2 = jnp.zeros((_L,), jnp.float32)
            a3 = jnp.zeros((_L,), jnp.float32)
            for c in range(0, _C // _L, 4):
                v0 = xbuf[par, r, pl.ds(c * _L, _L)]
                v1 = xbuf[par, r, pl.ds((c + 1) * _L, _L)]
                v2 = xbuf[par, r, pl.ds((c + 2) * _L, _L)]
                v3 = xbuf[par, r, pl.ds((c + 3) * _L, _L)]
                a0 = a0 + v0 * v0
                a1 = a1 + v1 * v1
                a2 = a2 + v2 * v2
                a3 = a3 + v3 * v3
            ssq = jnp.sum((a0 + a1) + (a2 + a3))
            sc = jnp.maximum(ssq, jnp.float32(1e-24))
            bits = lax.bitcast_convert_type(sc, jnp.int32)
            bits = jnp.int32(0x5F3759DF) - lax.shift_right_arithmetic(bits, 1)
            y = lax.bitcast_convert_type(bits, jnp.float32)
            half = sc * jnp.float32(0.5)
            for _ in range(4):
                y = y * (jnp.float32(1.5) - half * y * y)
            rinv = jnp.where(ssq >= jnp.float32(1e-24), y, jnp.float32(1e12))

            g0 = (r // _L) * _L
            sv = sbuf[pl.ds(t * _CHUNK + g0, _L)]
            sj = jnp.max(jnp.where(iota == (r - g0), sv, jnp.int32(-1)))
            for c in range(_C // _L):
                v = xbuf[par, r, pl.ds(c * _L, _L)]
                plsc.addupdate(acc.at[sj, pl.ds(c * _L, _L)], v * rinv)
            q = ssq * rinv * rinv
            ext = jnp.where(iota == 0, q,
                            jnp.where(iota == 1, jnp.float32(1.0),
                                      jnp.float32(0.0)))
            plsc.addupdate(acc.at[sj, pl.ds(_C, _L)], ext)

        return carry

    lax.fori_loop(0, _NCHUNK, _chunk, 0)
    pltpu.sync_copy(acc, out_hbm.at[wid])


def _make_sc_call(interpret=False):
    mesh = plsc.VectorSubcoreMesh(
        core_axis_name="c", subcore_axis_name="s",
        num_cores=_NC, num_subcores=_NS)
    return pl.kernel(
        _sc_body,
        out_type=jax.ShapeDtypeStruct((_NW, _K, _EXT), jnp.float32),
        mesh=mesh,
        compiler_params=pltpu.CompilerParams(needs_layout_passes=False),
        scratch_types=[
            pltpu.VMEM((2, _CHUNK, _C), jnp.float32),
            pltpu.VMEM((_RPW,), jnp.int32),
            pltpu.VMEM((_K, _EXT), jnp.float32),
            pltpu.SemaphoreType.DMA,
        ],
        interpret=interpret,
    )


_RB = 1024


def _tc_half_body(x_ref, s_ref, o_ref):
    x = x_ref[...]
    ssq = jnp.sum(x * x, axis=1, keepdims=True)
    safe = jnp.maximum(ssq, jnp.float32(1e-24))
    rinv = jnp.where(ssq >= jnp.float32(1e-24), lax.rsqrt(safe),
                     jnp.float32(1e12))
    rinv_row = jnp.transpose(rinv)
    ssq_row = jnp.transpose(ssq)
    scores = s_ref[...]
    lvl = lax.broadcasted_iota(jnp.int32, (_K, 1), 0)
    onehot = (lvl == scores).astype(jnp.float32)
    xs = x * rinv
    xs_hi = xs.astype(jnp.bfloat16)
    xs_lo = (xs - xs_hi.astype(jnp.float32)).astype(jnp.bfloat16)
    oh_bf = onehot.astype(jnp.bfloat16)
    S = (jax.lax.dot(oh_bf, xs_hi, preferred_element_type=jnp.float32)
         + jax.lax.dot(oh_bf, xs_lo, preferred_element_type=jnp.float32))
    q_row = ssq_row * rinv_row * rinv_row
    Qcol = jnp.sum(onehot * q_row, axis=1, keepdims=True)
    ncol = jnp.sum(onehot, axis=1, keepdims=True)
    part = jnp.concatenate(
        [S, Qcol, ncol, jnp.zeros((_K, _EXT - _C - 2), jnp.float32)], axis=1)

    @pl.when(pl.program_id(0) == 0)
    def _():
        o_ref[...] = part

    @pl.when(pl.program_id(0) > 0)
    def _():
        o_ref[...] += part


def _make_tc_half_call(interpret=False):
    off = _BSC // _RB
    return pl.pallas_call(
        _tc_half_body,
        grid=(_BTC // _RB,),
        in_specs=[
            pl.BlockSpec((_RB, _C), lambda i: (i + off, 0)),
            pl.BlockSpec((1, _RB), lambda i: (0, i + off)),
        ],
        out_specs=pl.BlockSpec((_K, _EXT), lambda i: (0, 0)),
        out_shape=jax.ShapeDtypeStruct((_K, _EXT), jnp.float32),
        interpret=interpret,
    )


def _combine_body(p_ref, t_ref, o_ref):
    S = jnp.sum(p_ref[...], axis=0) + t_ref[...]
    Sf = S[:, :_C]
    n = S[:, _C + 1:_C + 2]
    A = Sf[0:_K - 2]
    Bv = Sf[1:_K - 1]
    Cv = Sf[2:_K]
    dbc = jnp.sum(Bv * Cv, axis=1, keepdims=True)
    dac = jnp.sum(A * Cv, axis=1, keepdims=True)
    dab = jnp.sum(A * Bv, axis=1, keepdims=True)
    n1 = n[0:_K - 2]
    n2 = n[1:_K - 1]
    n3 = n[2:_K]
    Q2 = S[1:_K - 1, _C:_C + 1]
    one = jnp.float32(1.0)
    term = (dbc / jnp.maximum(n2 * n3, one)
            - Q2 / jnp.maximum(n2, one)
            - dac / jnp.maximum(n1 * n3, one)
            + dab / jnp.maximum(n1 * n2, one))
    valid = (n1 > 0) & (n2 > 0) & (n3 > 0)
    total = jnp.sum(jnp.where(valid, term, jnp.float32(0.0)))
    o_ref[0, 0] = total / jnp.float32(_K - 2)


def _make_combine_call(interpret=False):
    return pl.pallas_call(
        _combine_body,
        out_shape=jax.ShapeDtypeStruct((1, 1), jnp.float32),
        out_specs=pl.BlockSpec(memory_space=pltpu.SMEM),
        interpret=interpret,
    )


@jax.jit
def kernel(video_features, video_scores):
    partials = _make_sc_call()(video_features, video_scores)
    tc_part = _make_tc_half_call()(video_features,
                                   video_scores.reshape(1, _B))
    out = _make_combine_call()(partials, tc_part)
    return out[0, 0]


# speedup vs baseline: 1.0848x; 1.0848x over previous
import functools

import jax
import jax.numpy as jnp
from jax import lax
from jax.experimental import pallas as pl
from jax.experimental.pallas import tpu as pltpu
from jax.experimental.pallas import tpu_sc as plsc

_K = 16
_C = 512
_B = 8192
_NC = 2
_NS = 16
_NW = _NC * _NS
_BSC = 1536
_BTC = _B - _BSC
_RPW = _BSC // _NW
_CHUNK = 24
_NCHUNK = _RPW // _CHUNK
_EXT = _C + 16
_L = 16


def _sc_body(x_hbm, s_hbm, out_hbm, xbuf, sbuf, acc, sem):
    cid = lax.axis_index("c")
    sid = lax.axis_index("s")
    wid = sid * _NC + cid

    zeros = jnp.zeros((_L,), jnp.float32)
    iota = lax.iota(jnp.int32, _L)

    def _issue(t):
        r0 = wid * _RPW + t * _CHUNK
        pltpu.async_copy(x_hbm.at[pl.ds(r0, _CHUNK), :],
                         xbuf.at[lax.rem(t, 2)], sem)

    def _drain(t):
        r0 = wid * _RPW + t * _CHUNK
        pltpu.make_async_copy(x_hbm.at[pl.ds(r0, _CHUNK), :],
                              xbuf.at[lax.rem(t, 2)], sem).wait()

    _issue(jnp.int32(0))
    pltpu.sync_copy(s_hbm.at[pl.ds(wid * _RPW, _RPW)], sbuf)

    def _zero(i, carry):
        acc[lax.div(i, _EXT // _L), pl.ds(lax.rem(i, _EXT // _L) * _L, _L)] = zeros
        return carry

    lax.fori_loop(0, _K * (_EXT // _L), _zero, 0)

    def _chunk(t, carry):
        @pl.when(t + 1 < _NCHUNK)
        def _():
            _issue(t + 1)

        _drain(t)
        par = lax.rem(t, 2)

        @plsc.parallel_loop(0, _CHUNK, unroll=4)
        def _row(r):
            a0 = jnp.zeros((_L,), jnp.float32)
            a1 = jnp.zeros((_L,), jnp.float32)
            a2 = jnp.zeros((_L,), jnp.float32)
            a3 = jnp.zeros((_L,), jnp.float32)
            for c in range(0, _C // _L, 4):
                v0 = xbuf[par, r, pl.ds(c * _L, _L)]
                v1 = xbuf[par, r, pl.ds((c + 1) * _L, _L)]
                v2 = xbuf[par, r, pl.ds((c + 2) * _L, _L)]
                v3 = xbuf[par, r, pl.ds((c + 3) * _L, _L)]
                a0 = a0 + v0 * v0
                a1 = a1 + v1 * v1
                a2 = a2 + v2 * v2
                a3 = a3 + v3 * v3
            ssq = jnp.sum((a0 + a1) + (a2 + a3))
            sc = jnp.maximum(ssq, jnp.float32(1e-24))
            bits = lax.bitcast_convert_type(sc, jnp.int32)
            bits = jnp.int32(0x5F3759DF) - lax.shift_right_arithmetic(bits, 1)
            y = lax.bitcast_convert_type(bits, jnp.float32)
            half = sc * jnp.float32(0.5)
            for _ in range(4):
                y = y * (jnp.float32(1.5) - half * y * y)
            rinv = jnp.where(ssq >= jnp.float32(1e-24), y, jnp.float32(1e12))

            g0 = (r // _L) * _L
            sv = sbuf[pl.ds(t * _CHUNK + g0, _L)]
            sj = jnp.max(jnp.where(iota == (r - g0), sv, jnp.int32(-1)))
            for c in range(_C // _L):
                v = xbuf[par, r, pl.ds(c * _L, _L)]
                plsc.addupdate(acc.at[sj, pl.ds(c * _L, _L)], v * rinv)
            q = ssq * rinv * rinv
            ext = jnp.where(iota == 0, q,
                            jnp.where(iota == 1, jnp.float32(1.0),
                                      jnp.float32(0.0)))
            plsc.addupdate(acc.at[sj, pl.ds(_C, _L)], ext)

        return carry

    lax.fori_loop(0, _NCHUNK, _chunk, 0)
    pltpu.sync_copy(acc, out_hbm.at[wid])


def _make_sc_call(interpret=False):
    mesh = plsc.VectorSubcoreMesh(
        core_axis_name="c", subcore_axis_name="s",
        num_cores=_NC, num_subcores=_NS)
    return pl.kernel(
        _sc_body,
        out_type=jax.ShapeDtypeStruct((_NW, _K, _EXT), jnp.float32),
        mesh=mesh,
        compiler_params=pltpu.CompilerParams(needs_layout_passes=False),
        scratch_types=[
            pltpu.VMEM((2, _CHUNK, _C), jnp.float32),
            pltpu.VMEM((_RPW,), jnp.int32),
            pltpu.VMEM((_K, _EXT), jnp.float32),
            pltpu.SemaphoreType.DMA,
        ],
        interpret=interpret,
    )


_RB = 512


def _tc_half_body(x_ref, s_ref, o_ref):
    x = x_ref[...]
    ssq = jnp.sum(x * x, axis=1, keepdims=True)
    safe = jnp.maximum(ssq, jnp.float32(1e-24))
    rinv = jnp.where(ssq >= jnp.float32(1e-24), lax.rsqrt(safe),
                     jnp.float32(1e12))
    rinv_row = jnp.transpose(rinv)
    ssq_row = jnp.transpose(ssq)
    scores = s_ref[...]
    lvl = lax.broadcasted_iota(jnp.int32, (_K, 1), 0)
    onehot = (lvl == scores).astype(jnp.float32)
    xs = x * rinv
    xs_hi = xs.astype(jnp.bfloat16)
    xs_lo = (xs - xs_hi.astype(jnp.float32)).astype(jnp.bfloat16)
    oh_bf = onehot.astype(jnp.bfloat16)
    S = (jax.lax.dot(oh_bf, xs_hi, preferred_element_type=jnp.float32)
         + jax.lax.dot(oh_bf, xs_lo, preferred_element_type=jnp.float32))
    q_row = ssq_row * rinv_row * rinv_row
    Qcol = jnp.sum(onehot * q_row, axis=1, keepdims=True)
    ncol = jnp.sum(onehot, axis=1, keepdims=True)
    part = jnp.concatenate(
        [S, Qcol, ncol, jnp.zeros((_K, _EXT - _C - 2), jnp.float32)], axis=1)

    @pl.when(pl.program_id(0) == 0)
    def _():
        o_ref[...] = part

    @pl.when(pl.program_id(0) > 0)
    def _():
        o_ref[...] += part


def _make_tc_half_call(interpret=False):
    off = _BSC // _RB
    return pl.pallas_call(
        _tc_half_body,
        grid=(_BTC // _RB,),
        in_specs=[
            pl.BlockSpec((_RB, _C), lambda i: (i + off, 0)),
            pl.BlockSpec((1, _RB), lambda i: (0, i + off)),
        ],
        out_specs=pl.BlockSpec((_K, _EXT), lambda i: (0, 0)),
        out_shape=jax.ShapeDtypeStruct((_K, _EXT), jnp.float32),
        interpret=interpret,
    )


def _combine_body(p_ref, t_ref, o_ref):
    S = jnp.sum(p_ref[...], axis=0) + t_ref[...]
    Sf = S[:, :_C]
    n = S[:, _C + 1:_C + 2]
    A = Sf[0:_K - 2]
    Bv = Sf[1:_K - 1]
    Cv = Sf[2:_K]
    dbc = jnp.sum(Bv * Cv, axis=1, keepdims=True)
    dac = jnp.sum(A * Cv, axis=1, keepdims=True)
    dab = jnp.sum(A * Bv, axis=1, keepdims=True)
    n1 = n[0:_K - 2]
    n2 = n[1:_K - 1]
    n3 = n[2:_K]
    Q2 = S[1:_K - 1, _C:_C + 1]
    one = jnp.float32(1.0)
    term = (dbc / jnp.maximum(n2 * n3, one)
            - Q2 / jnp.maximum(n2, one)
            - dac / jnp.maximum(n1 * n3, one)
            + dab / jnp.maximum(n1 * n2, one))
    valid = (n1 > 0) & (n2 > 0) & (n3 > 0)
    total = jnp.sum(jnp.where(valid, term, jnp.float32(0.0)))
    o_ref[0, 0] = total / jnp.float32(_K - 2)


def _make_combine_call(interpret=False):
    return pl.pallas_call(
        _combine_body,
        out_shape=jax.ShapeDtypeStruct((1, 1), jnp.float32),
        out_specs=pl.BlockSpec(memory_space=pltpu.SMEM),
        interpret=interpret,
    )


@jax.jit
def kernel(video_features, video_scores):
    partials = _make_sc_call()(video_features, video_scores)
    tc_part = _make_tc_half_call()(video_features,
                                   video_scores.reshape(1, _B))
    out = _make_combine_call()(partials, tc_part)
    return out[0, 0]
